# Initial kernel scaffold; baseline (speedup 1.0000x reference)
import jax
import jax.numpy as jnp
from jax.experimental import pallas as pl


def kernel(hidden_states, weight):
    raise NotImplementedError("write your pallas kernel here")


# speedup vs baseline: 1.3536x; 1.3536x over previous
import functools

import jax
import jax.numpy as jnp
from jax.experimental import pallas as pl
from jax.experimental.pallas import tpu as pltpu

NUM_EXPERTS = 64
TOP_K = 2
HIDDEN = 2048
T = 16384

ROWS = 512


def _router_body(h_ref, w_ref, probs_ref, tw_ref, ti_ref):
    h = h_ref[...]
    w = w_ref[...]
    logits = jax.lax.dot_general(
        h, w,
        dimension_numbers=(((1,), (1,)), ((), ())),
        preferred_element_type=jnp.float32,
    )
    m = jnp.max(logits, axis=-1, keepdims=True)
    e = jnp.exp(logits - m)
    s = jnp.sum(e, axis=-1, keepdims=True)
    probs = e / s
    probs_ref[...] = probs

    lane = jax.lax.broadcasted_iota(jnp.int32, probs.shape, 1)
    m1 = jnp.max(probs, axis=-1, keepdims=True)
    i1 = jnp.min(jnp.where(probs == m1, lane, NUM_EXPERTS), axis=-1, keepdims=True)
    masked = jnp.where(lane == i1, -1.0, probs)
    m2 = jnp.max(masked, axis=-1, keepdims=True)
    i2 = jnp.min(jnp.where(masked == m2, lane, NUM_EXPERTS), axis=-1, keepdims=True)

    denom = m1 + m2 + 1e-9
    tw_ref[...] = jnp.concatenate([m1 / denom, m2 / denom], axis=-1)
    ti_ref[...] = jnp.concatenate([i1, i2], axis=-1)


@jax.jit
def _router(hidden_states, weight):
    grid = (T // ROWS,)
    return pl.pallas_call(
        _router_body,
        grid=grid,
        in_specs=[
            pl.BlockSpec((ROWS, HIDDEN), lambda i: (i, 0)),
            pl.BlockSpec((NUM_EXPERTS, HIDDEN), lambda i: (0, 0)),
        ],
        out_specs=[
            pl.BlockSpec((ROWS, NUM_EXPERTS), lambda i: (i, 0)),
            pl.BlockSpec((ROWS, TOP_K), lambda i: (i, 0)),
            pl.BlockSpec((ROWS, TOP_K), lambda i: (i, 0)),
        ],
        out_shape=[
            jax.ShapeDtypeStruct((T, NUM_EXPERTS), jnp.float32),
            jax.ShapeDtypeStruct((T, TOP_K), jnp.float32),
            jax.ShapeDtypeStruct((T, TOP_K), jnp.int32),
        ],
    )(hidden_states, weight)


def kernel(hidden_states, weight):
    probs, top_w, top_i = _router(hidden_states, weight)
    return probs, top_w.astype(hidden_states.dtype), top_i.astype(jnp.int64)


# speedup vs baseline: 1.5706x; 1.1603x over previous
import functools

import jax
import jax.numpy as jnp
from jax.experimental import pallas as pl
from jax.experimental.pallas import tpu as pltpu

NUM_EXPERTS = 64
TOP_K = 2
HIDDEN = 2048
T = 16384

ROWS = 1024


def _router_body(h_ref, w_ref, probs_ref, tw_ref, ti_ref):
    h = h_ref[...]
    w = w_ref[...]
    logits = jax.lax.dot_general(
        h, w,
        dimension_numbers=(((1,), (1,)), ((), ())),
        preferred_element_type=jnp.float32,
    )
    m = jnp.max(logits, axis=-1, keepdims=True)
    e = jnp.exp(logits - m)
    s = jnp.sum(e, axis=-1, keepdims=True)
    probs = e / s
    probs_ref[...] = probs

    lane = jax.lax.broadcasted_iota(jnp.int32, probs.shape, 1)
    m1 = jnp.max(probs, axis=-1, keepdims=True)
    i1 = jnp.min(jnp.where(probs == m1, lane, NUM_EXPERTS), axis=-1, keepdims=True)
    masked = jnp.where(lane == i1, -1.0, probs)
    m2 = jnp.max(masked, axis=-1, keepdims=True)
    i2 = jnp.min(jnp.where(masked == m2, lane, NUM_EXPERTS), axis=-1, keepdims=True)

    denom = m1 + m2 + 1e-9
    tw_ref[...] = jnp.concatenate([m1 / denom, m2 / denom], axis=-1)
    ti_ref[...] = jnp.concatenate([i1, i2], axis=-1)


@jax.jit
def _router(hidden_states, weight):
    grid = (T // ROWS,)
    return pl.pallas_call(
        _router_body,
        grid=grid,
        in_specs=[
            pl.BlockSpec((ROWS, HIDDEN), lambda i: (i, 0)),
            pl.BlockSpec((NUM_EXPERTS, HIDDEN), lambda i: (0, 0)),
        ],
        out_specs=[
            pl.BlockSpec((ROWS, NUM_EXPERTS), lambda i: (i, 0)),
            pl.BlockSpec((ROWS, TOP_K), lambda i: (i, 0)),
            pl.BlockSpec((ROWS, TOP_K), lambda i: (i, 0)),
        ],
        out_shape=[
            jax.ShapeDtypeStruct((T, NUM_EXPERTS), jnp.float32),
            jax.ShapeDtypeStruct((T, TOP_K), jnp.float32),
            jax.ShapeDtypeStruct((T, TOP_K), jnp.int32),
        ],
    )(hidden_states, weight)


def kernel(hidden_states, weight):
    probs, top_w, top_i = _router(hidden_states, weight)
    return probs, top_w.astype(hidden_states.dtype), top_i.astype(jnp.int64)


# speedup vs baseline: 1.6306x; 1.0382x over previous
import functools

import jax
import jax.numpy as jnp
from jax.experimental import pallas as pl
from jax.experimental.pallas import tpu as pltpu

NUM_EXPERTS = 64
TOP_K = 2
HIDDEN = 2048
T = 16384

ROWS = 2048


def _router_body(h_ref, w_ref, probs_ref, tw_ref, ti_ref):
    h = h_ref[...]
    w = w_ref[...]
    logits = jax.lax.dot_general(
        h, w,
        dimension_numbers=(((1,), (1,)), ((), ())),
        preferred_element_type=jnp.float32,
    )
    m = jnp.max(logits, axis=-1, keepdims=True)
    e = jnp.exp(logits - m)
    s = jnp.sum(e, axis=-1, keepdims=True)
    probs = e / s
    probs_ref[...] = probs

    lane = jax.lax.broadcasted_iota(jnp.int32, probs.shape, 1)
    m1 = jnp.max(probs, axis=-1, keepdims=True)
    i1 = jnp.min(jnp.where(probs == m1, lane, NUM_EXPERTS), axis=-1, keepdims=True)
    masked = jnp.where(lane == i1, -1.0, probs)
    m2 = jnp.max(masked, axis=-1, keepdims=True)
    i2 = jnp.min(jnp.where(masked == m2, lane, NUM_EXPERTS), axis=-1, keepdims=True)

    denom = m1 + m2 + 1e-9
    tw_ref[...] = jnp.concatenate([m1 / denom, m2 / denom], axis=-1)
    ti_ref[...] = jnp.concatenate([i1, i2], axis=-1)


@jax.jit
def _router(hidden_states, weight):
    grid = (T // ROWS,)
    return pl.pallas_call(
        _router_body,
        grid=grid,
        in_specs=[
            pl.BlockSpec((ROWS, HIDDEN), lambda i: (i, 0)),
            pl.BlockSpec((NUM_EXPERTS, HIDDEN), lambda i: (0, 0)),
        ],
        out_specs=[
            pl.BlockSpec((ROWS, NUM_EXPERTS), lambda i: (i, 0)),
            pl.BlockSpec((ROWS, TOP_K), lambda i: (i, 0)),
            pl.BlockSpec((ROWS, TOP_K), lambda i: (i, 0)),
        ],
        out_shape=[
            jax.ShapeDtypeStruct((T, NUM_EXPERTS), jnp.float32),
            jax.ShapeDtypeStruct((T, TOP_K), jnp.float32),
            jax.ShapeDtypeStruct((T, TOP_K), jnp.int32),
        ],
    )(hidden_states, weight)


def kernel(hidden_states, weight):
    probs, top_w, top_i = _router(hidden_states, weight)
    return probs, top_w.astype(hidden_states.dtype), top_i.astype(jnp.int64)
